# Initial kernel scaffold; baseline (speedup 1.0000x reference)
import jax
import jax.numpy as jnp
from jax.experimental import pallas as pl


def kernel(x, edge_index, edge_attr, W_node, w_att, bias):
    raise NotImplementedError("write your pallas kernel here")


# speedup vs baseline: 7.8735x; 7.8735x over previous
import dataclasses
import functools

import jax
import jax.numpy as jnp
from jax import lax
from jax.experimental import pallas as pl
from jax.experimental.pallas import tpu as pltpu
from jax.experimental.pallas import tpu_sc as plsc

N_PAD = 10240
E_PAD = 327680
K = 128
CHUNKS = E_PAD // (32 * K)
STRIPE = N_PAD // 16


def _prep_body(x_ref, w_ref, wac_ref, ea_ref, we_ref, xw_ref, ac_ref, b_ref):
    xw = jnp.dot(x_ref[...], w_ref[...], preferred_element_type=jnp.float32)
    xw_ref[...] = xw
    ac_ref[...] = jnp.dot(xw, wac_ref[...], preferred_element_type=jnp.float32)
    b_ref[...] = jnp.dot(ea_ref[...], we_ref[...],
                         preferred_element_type=jnp.float32)


def _combine_body(pa_ref, pd_ref, bias_ref, o_ref):
    s = pa_ref[0] + pa_ref[1]
    d = pd_ref[0] + pd_ref[1] + 1e-16
    o_ref[...] = s / d[:, None] + bias_ref[...]


def _sc_body(n_edges,
             xw_hbm, a_hbm, c_hbm, b_hbm, dst_hbm, src_hbm,
             aggr_out, den_out,
             a_buf, c_buf, idx_d, idx_s, b_buf, g_buf, rows,
             aggr_sh, den_sh, sem):
    cid = lax.axis_index("c")
    sid = lax.axis_index("s")
    wid = sid * 2 + cid
    zeros16 = jnp.zeros((16,), jnp.float32)

    @pl.loop(0, K // 16)
    def _(i):
        g_buf[pl.ds(i * 16, 16)] = zeros16

    @pl.loop(0, K)
    def _(r):
        for cc in range(8):
            rows[r, pl.ds(cc * 16, 16)] = zeros16

    off = sid * STRIPE
    for i in range(STRIPE // K):
        pltpu.sync_copy(g_buf, den_sh.at[pl.ds(off + i * K, K)])
        pltpu.sync_copy(rows, aggr_sh.at[pl.ds(off + i * K, K)])

    pltpu.sync_copy(a_hbm, a_buf)
    pltpu.sync_copy(c_hbm, c_buf)
    plsc.subcore_barrier()

    base0 = wid * (CHUNKS * K)

    @pl.loop(0, CHUNKS)
    def _(t):
        base = base0 + t * K
        pltpu.sync_copy(dst_hbm.at[pl.ds(base, K)], idx_d)
        pltpu.sync_copy(src_hbm.at[pl.ds(base, K)], idx_s)
        pltpu.sync_copy(b_hbm.at[pl.ds(base, K)], b_buf)
        gather = pltpu.async_copy(xw_hbm.at[idx_s], rows, sem)

        @pl.loop(0, K // 16)
        def _(j):
            sl = pl.ds(j * 16, 16)
            d16 = idx_d[sl]
            s16 = idx_s[sl]
            av = plsc.load_gather(a_buf, [d16])
            cv = plsc.load_gather(c_buf, [s16])
            al = av + cv + b_buf[sl]
            al = jnp.where(al > 0, al, al * 0.2)
            al = jnp.minimum(al, 60.0)
            g = jnp.exp(al)
            eid = base + j * 16 + lax.iota(jnp.int32, 16)
            g_buf[sl] = jnp.where(eid < n_edges, g, 0.0)

        pltpu.sync_copy(g_buf, den_sh.at[idx_d], add=True)
        gather.wait()

        @pl.loop(0, K // 16)
        def _(j):
            g16 = g_buf[pl.ds(j * 16, 16)]
            for l in range(16):
                r = j * 16 + l
                gs = g16[l]
                for cc in range(8):
                    sl2 = pl.ds(cc * 16, 16)
                    rows[r, sl2] = rows[r, sl2] * gs

        pltpu.sync_copy(rows, aggr_sh.at[idx_d], add=True)

    plsc.subcore_barrier()
    pltpu.sync_copy(aggr_sh.at[pl.ds(off, STRIPE)],
                    aggr_out.at[cid, pl.ds(off, STRIPE)])
    pltpu.sync_copy(den_sh.at[pl.ds(off, STRIPE)],
                    den_out.at[cid, pl.ds(off, STRIPE)])


def kernel(x, edge_index, edge_attr, W_node, w_att, bias):
    N, D = x.shape
    E, De = edge_attr.shape
    f32 = jnp.float32

    dst = jnp.pad(edge_index[1].astype(jnp.int32), (0, E_PAD - E))
    src = jnp.pad(edge_index[0].astype(jnp.int32), (0, E_PAD - E))
    x_pad = jnp.pad(x, ((0, N_PAD - N), (0, 0)))
    ea_pad = jnp.pad(edge_attr, ((0, E_PAD - E), (0, 0)))
    wac = jnp.stack([w_att[:D], w_att[D + De:]], axis=1)
    we = w_att[D:D + De].reshape(De, 1)

    nb = 20
    xb = N_PAD // nb
    eb = E_PAD // nb
    xw, ac, b2 = pl.pallas_call(
        _prep_body,
        grid=(nb,),
        in_specs=[
            pl.BlockSpec((xb, D), lambda i: (i, 0)),
            pl.BlockSpec((D, D), lambda i: (0, 0)),
            pl.BlockSpec((D, 2), lambda i: (0, 0)),
            pl.BlockSpec((eb, De), lambda i: (i, 0)),
            pl.BlockSpec((De, 1), lambda i: (0, 0)),
        ],
        out_specs=[
            pl.BlockSpec((xb, D), lambda i: (i, 0)),
            pl.BlockSpec((xb, 2), lambda i: (i, 0)),
            pl.BlockSpec((eb, 1), lambda i: (i, 0)),
        ],
        out_shape=[
            jax.ShapeDtypeStruct((N_PAD, D), f32),
            jax.ShapeDtypeStruct((N_PAD, 2), f32),
            jax.ShapeDtypeStruct((E_PAD, 1), f32),
        ],
    )(x_pad, W_node, wac, ea_pad, we)

    a = ac[:, 0]
    c = ac[:, 1]
    b = b2[:, 0]

    mesh = plsc.VectorSubcoreMesh(core_axis_name="c", subcore_axis_name="s")
    cp = pltpu.CompilerParams()
    if "needs_layout_passes" in pltpu.CompilerParams.__dataclass_fields__:
        cp = dataclasses.replace(cp, needs_layout_passes=False)
    sc = pl.kernel(
        functools.partial(_sc_body, E),
        out_type=[
            jax.ShapeDtypeStruct((2, N_PAD, D), f32),
            jax.ShapeDtypeStruct((2, N_PAD), f32),
        ],
        mesh=mesh,
        scratch_types=[
            pltpu.VMEM((N_PAD,), f32),
            pltpu.VMEM((N_PAD,), f32),
            pltpu.VMEM((K,), jnp.int32),
            pltpu.VMEM((K,), jnp.int32),
            pltpu.VMEM((K,), f32),
            pltpu.VMEM((K,), f32),
            pltpu.VMEM((K, D), f32),
            pltpu.VMEM_SHARED((N_PAD, D), f32),
            pltpu.VMEM_SHARED((N_PAD,), f32),
            pltpu.SemaphoreType.DMA,
        ],
        compiler_params=cp,
    )
    pa, pd = sc(xw, a, c, b, dst, src)

    out = pl.pallas_call(
        _combine_body,
        grid=(nb,),
        in_specs=[
            pl.BlockSpec((2, xb, D), lambda i: (0, i, 0)),
            pl.BlockSpec((2, xb), lambda i: (0, i)),
            pl.BlockSpec((1, D), lambda i: (0, 0)),
        ],
        out_specs=pl.BlockSpec((xb, D), lambda i: (i, 0)),
        out_shape=jax.ShapeDtypeStruct((N_PAD, D), f32),
    )(pa, pd, bias.reshape(1, D))

    return out[:N]


# speedup vs baseline: 10.1068x; 1.2836x over previous
import dataclasses
import functools

import jax
import jax.numpy as jnp
from jax import lax
from jax.experimental import pallas as pl
from jax.experimental.pallas import tpu as pltpu
from jax.experimental.pallas import tpu_sc as plsc

N_PAD = 10240
K = 96
CHUNKS = 106
E_PAD = 32 * CHUNKS * K
E_ALLOC = 327680
STRIPE = N_PAD // 16


def _prep_body(x_ref, w_ref, wac_ref, ea_ref, we_ref, xw_ref, ac_ref, b_ref):
    xw = jnp.dot(x_ref[...], w_ref[...], preferred_element_type=jnp.float32)
    xw_ref[...] = xw
    ac_ref[...] = jnp.dot(xw, wac_ref[...], preferred_element_type=jnp.float32)
    b_ref[...] = jnp.dot(ea_ref[...], we_ref[...],
                         preferred_element_type=jnp.float32)


def _combine_body(pa_ref, pd_ref, bias_ref, o_ref):
    s = pa_ref[0] + pa_ref[1]
    d = pd_ref[0] + pd_ref[1] + 1e-16
    o_ref[...] = s / d[:, None] + bias_ref[...]


def _sc_body(n_edges,
             xw_hbm, a_hbm, c_hbm, b_hbm, dst_hbm, src_hbm,
             aggr_out, den_out,
             a_buf, c_buf,
             idx_d0, idx_d1, idx_s0, idx_s1, b0, b1, g0, g1, sidx0, sidx1,
             rows0, rows1,
             aggr_sh, den_sh,
             sem_in0, sem_in1, sem_g0, sem_g1, sem_s0, sem_s1):
    cid = lax.axis_index("c")
    sid = lax.axis_index("s")
    wid = sid * 2 + cid
    zeros16 = jnp.zeros((16,), jnp.float32)
    izeros16 = jnp.zeros((16,), jnp.int32)
    idx_d = (idx_d0, idx_d1)
    idx_s = (idx_s0, idx_s1)
    bb = (b0, b1)
    gb = (g0, g1)
    sidx = (sidx0, sidx1)
    rows = (rows0, rows1)
    sem_in = (sem_in0, sem_in1)
    sem_g = (sem_g0, sem_g1)
    sem_s = (sem_s0, sem_s1)

    @pl.loop(0, K // 16)
    def _(i):
        sl = pl.ds(i * 16, 16)
        for q in (0, 1):
            gb[q][sl] = zeros16
            idx_d[q][sl] = izeros16
            idx_s[q][sl] = izeros16
            sidx[q][sl] = izeros16

    @pl.loop(0, K)
    def _(r):
        for cc in range(8):
            sl = pl.ds(cc * 16, 16)
            rows0[r, sl] = zeros16
            rows1[r, sl] = zeros16

    off = sid * STRIPE
    for i in range(STRIPE // K):
        pltpu.sync_copy(gb[0], den_sh.at[pl.ds(off + i * K, K)])
        pltpu.sync_copy(rows0, aggr_sh.at[pl.ds(off + i * K, K)])
    rem = STRIPE - (STRIPE // K) * K
    if rem:
        pltpu.sync_copy(gb[0].at[pl.ds(0, rem)],
                        den_sh.at[pl.ds(off + STRIPE - rem, rem)])
        pltpu.sync_copy(rows0.at[pl.ds(0, rem)],
                        aggr_sh.at[pl.ds(off + STRIPE - rem, rem)])

    pltpu.sync_copy(a_hbm, a_buf)
    pltpu.sync_copy(c_hbm, c_buf)
    plsc.subcore_barrier()

    base0 = wid * (CHUNKS * K)

    def issue_in(q, t):
        base = base0 + t * K
        pltpu.async_copy(dst_hbm.at[pl.ds(base, K)], idx_d[q], sem_in[q])
        pltpu.async_copy(src_hbm.at[pl.ds(base, K)], idx_s[q], sem_in[q])
        pltpu.async_copy(b_hbm.at[pl.ds(base, K)], bb[q], sem_in[q])

    def wait_in(q):
        pltpu.make_async_copy(dst_hbm.at[pl.ds(0, K)], idx_d[q], sem_in[q]).wait()
        pltpu.make_async_copy(src_hbm.at[pl.ds(0, K)], idx_s[q], sem_in[q]).wait()
        pltpu.make_async_copy(b_hbm.at[pl.ds(0, K)], bb[q], sem_in[q]).wait()

    def wait_g(q):
        pltpu.make_async_copy(xw_hbm.at[idx_s[q]], rows[q], sem_g[q]).wait()

    def wait_s(q):
        pltpu.make_async_copy(rows[q], aggr_sh.at[sidx[q]], sem_s[q]).wait()

    pltpu.async_copy(rows1, aggr_sh.at[sidx1], sem_s1, add=True)
    issue_in(0, 0)
    issue_in(1, 1)
    wait_in(0)
    pltpu.async_copy(xw_hbm.at[idx_s0], rows0, sem_g0)

    def phase(q, qn, t):
        base = base0 + t * K

        @pl.loop(0, K // 16)
        def _(j):
            sl = pl.ds(j * 16, 16)
            d16 = idx_d[q][sl]
            s16 = idx_s[q][sl]
            av = plsc.load_gather(a_buf, [d16])
            cv = plsc.load_gather(c_buf, [s16])
            al = av + cv + bb[q][sl]
            al = jnp.where(al > 0, al, al * 0.2)
            al = jnp.minimum(al, 60.0)
            g = jnp.exp(al)
            eid = base + j * 16 + lax.iota(jnp.int32, 16)
            gb[q][sl] = jnp.where(eid < n_edges, g, 0.0)
            sidx[q][sl] = d16

        pltpu.sync_copy(gb[q], den_sh.at[sidx[q]], add=True)
        wait_g(q)
        issue_in(q, t + 2)

        @pl.loop(0, K // 16)
        def _(j):
            g16 = gb[q][pl.ds(j * 16, 16)]
            for l in range(16):
                r = j * 16 + l
                gs = g16[l]
                for cc in range(8):
                    sl2 = pl.ds(cc * 16, 16)
                    rows[q][r, sl2] = rows[q][r, sl2] * gs

        pltpu.async_copy(rows[q], aggr_sh.at[sidx[q]], sem_s[q], add=True)
        wait_in(qn)
        wait_s(qn)
        pltpu.async_copy(xw_hbm.at[idx_s[qn]], rows[qn], sem_g[qn])

    @pl.loop(0, CHUNKS // 2)
    def _(i):
        phase(0, 1, 2 * i)
        phase(1, 0, 2 * i + 1)

    wait_in(1)
    wait_g(0)
    wait_s(1)

    plsc.subcore_barrier()
    pltpu.sync_copy(aggr_sh.at[pl.ds(off, STRIPE)],
                    aggr_out.at[cid, pl.ds(off, STRIPE)])
    pltpu.sync_copy(den_sh.at[pl.ds(off, STRIPE)],
                    den_out.at[cid, pl.ds(off, STRIPE)])


def kernel(x, edge_index, edge_attr, W_node, w_att, bias):
    N, D = x.shape
    E, De = edge_attr.shape
    f32 = jnp.float32

    dst = jnp.pad(edge_index[1].astype(jnp.int32), (0, E_ALLOC - E))
    src = jnp.pad(edge_index[0].astype(jnp.int32), (0, E_ALLOC - E))
    x_pad = jnp.pad(x, ((0, N_PAD - N), (0, 0)))
    ea_pad = jnp.pad(edge_attr, ((0, E_ALLOC - E), (0, 0)))
    wac = jnp.stack([w_att[:D], w_att[D + De:]], axis=1)
    we = w_att[D:D + De].reshape(De, 1)

    nb = 20
    xb = N_PAD // nb
    eb = E_ALLOC // nb
    xw, ac, b2 = pl.pallas_call(
        _prep_body,
        grid=(nb,),
        in_specs=[
            pl.BlockSpec((xb, D), lambda i: (i, 0)),
            pl.BlockSpec((D, D), lambda i: (0, 0)),
            pl.BlockSpec((D, 2), lambda i: (0, 0)),
            pl.BlockSpec((eb, De), lambda i: (i, 0)),
            pl.BlockSpec((De, 1), lambda i: (0, 0)),
        ],
        out_specs=[
            pl.BlockSpec((xb, D), lambda i: (i, 0)),
            pl.BlockSpec((xb, 2), lambda i: (i, 0)),
            pl.BlockSpec((eb, 1), lambda i: (i, 0)),
        ],
        out_shape=[
            jax.ShapeDtypeStruct((N_PAD, D), f32),
            jax.ShapeDtypeStruct((N_PAD, 2), f32),
            jax.ShapeDtypeStruct((E_ALLOC, 1), f32),
        ],
    )(x_pad, W_node, wac, ea_pad, we)

    a = ac[:, 0]
    c = ac[:, 1]
    b = b2[:, 0]

    mesh = plsc.VectorSubcoreMesh(core_axis_name="c", subcore_axis_name="s")
    cp = pltpu.CompilerParams()
    if "needs_layout_passes" in pltpu.CompilerParams.__dataclass_fields__:
        cp = dataclasses.replace(cp, needs_layout_passes=False)
    sc = pl.kernel(
        functools.partial(_sc_body, E),
        out_type=[
            jax.ShapeDtypeStruct((2, N_PAD, D), f32),
            jax.ShapeDtypeStruct((2, N_PAD), f32),
        ],
        mesh=mesh,
        scratch_types=[
            pltpu.VMEM((N_PAD,), f32),
            pltpu.VMEM((N_PAD,), f32),
            pltpu.VMEM((K,), jnp.int32),
            pltpu.VMEM((K,), jnp.int32),
            pltpu.VMEM((K,), jnp.int32),
            pltpu.VMEM((K,), jnp.int32),
            pltpu.VMEM((K,), f32),
            pltpu.VMEM((K,), f32),
            pltpu.VMEM((K,), f32),
            pltpu.VMEM((K,), f32),
            pltpu.VMEM((K,), jnp.int32),
            pltpu.VMEM((K,), jnp.int32),
            pltpu.VMEM((K, D), f32),
            pltpu.VMEM((K, D), f32),
            pltpu.VMEM_SHARED((N_PAD, D), f32),
            pltpu.VMEM_SHARED((N_PAD,), f32),
            pltpu.SemaphoreType.DMA,
            pltpu.SemaphoreType.DMA,
            pltpu.SemaphoreType.DMA,
            pltpu.SemaphoreType.DMA,
            pltpu.SemaphoreType.DMA,
            pltpu.SemaphoreType.DMA,
        ],
        compiler_params=cp,
    )
    pa, pd = sc(xw, a, c, b, dst, src)

    out = pl.pallas_call(
        _combine_body,
        grid=(nb,),
        in_specs=[
            pl.BlockSpec((2, xb, D), lambda i: (0, i, 0)),
            pl.BlockSpec((2, xb), lambda i: (0, i)),
            pl.BlockSpec((1, D), lambda i: (0, 0)),
        ],
        out_specs=pl.BlockSpec((xb, D), lambda i: (i, 0)),
        out_shape=jax.ShapeDtypeStruct((N_PAD, D), f32),
    )(pa, pd, bias.reshape(1, D))

    return out[:N]


# speedup vs baseline: 12.6157x; 1.2482x over previous
import dataclasses
import functools

import jax
import jax.numpy as jnp
from jax import lax
from jax.experimental import pallas as pl
from jax.experimental.pallas import tpu as pltpu
from jax.experimental.pallas import tpu_sc as plsc

N_PAD = 10240
K = 96
CHUNKS = 106
E_PAD = 32 * CHUNKS * K
E_ALLOC = 327680
STRIPE = N_PAD // 16


def _prep_body(x_ref, w_ref, wac_ref, ea_ref, wbig_ref, xw_ref, ac_ref, b_ref):
    xw = jnp.dot(x_ref[...], w_ref[...], preferred_element_type=jnp.float32)
    xw_ref[...] = xw
    ac_ref[...] = jnp.dot(xw, wac_ref[...], preferred_element_type=jnp.float32)
    b_ref[...] = jnp.dot(ea_ref[...], wbig_ref[...],
                         preferred_element_type=jnp.float32)


def _combine_body(pa_ref, pd_ref, bias_ref, o_ref):
    s = pa_ref[0] + pa_ref[1]
    d = pd_ref[0] + pd_ref[1] + 1e-16
    o_ref[...] = s / d[:, None] + bias_ref[...]


def _sc_body(n_edges,
             xw_hbm, a_hbm, c_hbm, b_hbm, dst_hbm, src_hbm,
             aggr_out, den_out,
             a_buf, c_buf,
             idx_d0, idx_d1, idx_s0, idx_s1, b0, b1, g0, g1, sidx0, sidx1,
             rows0, rows1,
             aggr_sh, den_sh,
             sem_in0, sem_in1, sem_g0, sem_g1, sem_s0, sem_s1):
    cid = lax.axis_index("c")
    sid = lax.axis_index("s")
    wid = sid * 2 + cid
    zeros16 = jnp.zeros((16,), jnp.float32)
    izeros16 = jnp.zeros((16,), jnp.int32)
    idx_d = (idx_d0, idx_d1)
    idx_s = (idx_s0, idx_s1)
    bb = (b0, b1)
    gb = (g0, g1)
    sidx = (sidx0, sidx1)
    rows = (rows0, rows1)
    sem_in = (sem_in0, sem_in1)
    sem_g = (sem_g0, sem_g1)
    sem_s = (sem_s0, sem_s1)

    @pl.loop(0, K // 16)
    def _(i):
        sl = pl.ds(i * 16, 16)
        for q in (0, 1):
            gb[q][sl] = zeros16
            idx_d[q][sl] = izeros16
            idx_s[q][sl] = izeros16
            sidx[q][sl] = izeros16

    @pl.loop(0, K)
    def _(r):
        for cc in range(8):
            sl = pl.ds(cc * 16, 16)
            rows0[r, sl] = zeros16
            rows1[r, sl] = zeros16

    off = sid * STRIPE
    for i in range(STRIPE // K):
        pltpu.sync_copy(gb[0], den_sh.at[pl.ds(off + i * K, K)])
        pltpu.sync_copy(rows0, aggr_sh.at[pl.ds(off + i * K, K)])
    rem = STRIPE - (STRIPE // K) * K
    if rem:
        pltpu.sync_copy(gb[0].at[pl.ds(0, rem)],
                        den_sh.at[pl.ds(off + STRIPE - rem, rem)])
        pltpu.sync_copy(rows0.at[pl.ds(0, rem)],
                        aggr_sh.at[pl.ds(off + STRIPE - rem, rem)])

    pltpu.sync_copy(a_hbm, a_buf)
    pltpu.sync_copy(c_hbm, c_buf)
    plsc.subcore_barrier()

    base0 = wid * (CHUNKS * K)

    def issue_in(q, t):
        base = base0 + t * K
        pltpu.async_copy(dst_hbm.at[pl.ds(base, K)], idx_d[q], sem_in[q])
        pltpu.async_copy(src_hbm.at[pl.ds(base, K)], idx_s[q], sem_in[q])
        pltpu.async_copy(b_hbm.at[pl.ds(base, K)], bb[q], sem_in[q])

    def wait_in(q):
        pltpu.make_async_copy(dst_hbm.at[pl.ds(0, K)], idx_d[q], sem_in[q]).wait()
        pltpu.make_async_copy(src_hbm.at[pl.ds(0, K)], idx_s[q], sem_in[q]).wait()
        pltpu.make_async_copy(b_hbm.at[pl.ds(0, K)], bb[q], sem_in[q]).wait()

    def wait_g(q):
        pltpu.make_async_copy(xw_hbm.at[idx_s[q]], rows[q], sem_g[q]).wait()

    def wait_s(q):
        pltpu.make_async_copy(rows[q], aggr_sh.at[sidx[q]], sem_s[q]).wait()

    pltpu.async_copy(rows1, aggr_sh.at[sidx1], sem_s1, add=True)
    issue_in(0, 0)
    issue_in(1, 1)
    wait_in(0)
    pltpu.async_copy(xw_hbm.at[idx_s0], rows0, sem_g0)

    def phase(q, qn, t):
        base = base0 + t * K

        @pl.loop(0, K // 16)
        def _(j):
            sl = pl.ds(j * 16, 16)
            d16 = idx_d[q][sl]
            s16 = idx_s[q][sl]
            av = plsc.load_gather(a_buf, [d16])
            cv = plsc.load_gather(c_buf, [s16])
            al = av + cv + bb[q][sl]
            al = jnp.where(al > 0, al, al * 0.2)
            al = jnp.minimum(al, 60.0)
            g = jnp.exp(al)
            eid = base + j * 16 + lax.iota(jnp.int32, 16)
            gb[q][sl] = jnp.where(eid < n_edges, g, 0.0)
            sidx[q][sl] = d16

        pltpu.sync_copy(gb[q], den_sh.at[sidx[q]], add=True)
        wait_g(q)
        issue_in(q, t + 2)

        @pl.loop(0, K // 16)
        def _(j):
            g16 = gb[q][pl.ds(j * 16, 16)]
            for l in range(16):
                r = j * 16 + l
                gs = g16[l]
                for cc in range(8):
                    sl2 = pl.ds(cc * 16, 16)
                    rows[q][r, sl2] = rows[q][r, sl2] * gs

        pltpu.async_copy(rows[q], aggr_sh.at[sidx[q]], sem_s[q], add=True)
        wait_in(qn)
        wait_s(qn)
        pltpu.async_copy(xw_hbm.at[idx_s[qn]], rows[qn], sem_g[qn])

    @pl.loop(0, CHUNKS // 2)
    def _(i):
        phase(0, 1, 2 * i)
        phase(1, 0, 2 * i + 1)

    wait_in(1)
    wait_g(0)
    wait_s(1)

    plsc.subcore_barrier()
    pltpu.sync_copy(aggr_sh.at[pl.ds(off, STRIPE)],
                    aggr_out.at[cid, pl.ds(off, STRIPE)])
    pltpu.sync_copy(den_sh.at[pl.ds(off, STRIPE)],
                    den_out.at[cid, pl.ds(off, STRIPE)])


def kernel(x, edge_index, edge_attr, W_node, w_att, bias):
    N, D = x.shape
    E, De = edge_attr.shape
    f32 = jnp.float32

    dst = jnp.pad(edge_index[1].astype(jnp.int32), (0, E_ALLOC - E))
    src = jnp.pad(edge_index[0].astype(jnp.int32), (0, E_ALLOC - E))
    wac = jnp.stack([w_att[:D], w_att[D + De:]], axis=1)
    npack = D // De
    blkmask = (jnp.arange(D)[:, None] // De) == jnp.arange(npack)[None, :]
    wbig = jnp.where(blkmask, jnp.tile(w_att[D:D + De], npack)[:, None], 0.0)
    ea_r = edge_attr.reshape(E // npack, D)

    nb = 20
    xb = N_PAD // nb
    eb = E_ALLOC // npack // nb
    xw, ac, b2 = pl.pallas_call(
        _prep_body,
        grid=(nb,),
        in_specs=[
            pl.BlockSpec((xb, D), lambda i: (i, 0)),
            pl.BlockSpec((D, D), lambda i: (0, 0)),
            pl.BlockSpec((D, 2), lambda i: (0, 0)),
            pl.BlockSpec((eb, D), lambda i: (i, 0)),
            pl.BlockSpec((D, npack), lambda i: (0, 0)),
        ],
        out_specs=[
            pl.BlockSpec((xb, D), lambda i: (i, 0)),
            pl.BlockSpec((xb, 2), lambda i: (i, 0)),
            pl.BlockSpec((eb, npack), lambda i: (i, 0)),
        ],
        out_shape=[
            jax.ShapeDtypeStruct((N_PAD, D), f32),
            jax.ShapeDtypeStruct((N_PAD, 2), f32),
            jax.ShapeDtypeStruct((E_ALLOC // npack, npack), f32),
        ],
    )(x, W_node, wac, ea_r, wbig)

    a = ac[:, 0]
    c = ac[:, 1]
    b = b2.reshape(E_ALLOC)

    mesh = plsc.VectorSubcoreMesh(core_axis_name="c", subcore_axis_name="s")
    cp = pltpu.CompilerParams()
    if "needs_layout_passes" in pltpu.CompilerParams.__dataclass_fields__:
        cp = dataclasses.replace(cp, needs_layout_passes=False)
    sc = pl.kernel(
        functools.partial(_sc_body, E),
        out_type=[
            jax.ShapeDtypeStruct((2, N_PAD, D), f32),
            jax.ShapeDtypeStruct((2, N_PAD), f32),
        ],
        mesh=mesh,
        scratch_types=[
            pltpu.VMEM((N_PAD,), f32),
            pltpu.VMEM((N_PAD,), f32),
            pltpu.VMEM((K,), jnp.int32),
            pltpu.VMEM((K,), jnp.int32),
            pltpu.VMEM((K,), jnp.int32),
            pltpu.VMEM((K,), jnp.int32),
            pltpu.VMEM((K,), f32),
            pltpu.VMEM((K,), f32),
            pltpu.VMEM((K,), f32),
            pltpu.VMEM((K,), f32),
            pltpu.VMEM((K,), jnp.int32),
            pltpu.VMEM((K,), jnp.int32),
            pltpu.VMEM((K, D), f32),
            pltpu.VMEM((K, D), f32),
            pltpu.VMEM_SHARED((N_PAD, D), f32),
            pltpu.VMEM_SHARED((N_PAD,), f32),
            pltpu.SemaphoreType.DMA,
            pltpu.SemaphoreType.DMA,
            pltpu.SemaphoreType.DMA,
            pltpu.SemaphoreType.DMA,
            pltpu.SemaphoreType.DMA,
            pltpu.SemaphoreType.DMA,
        ],
        compiler_params=cp,
    )
    pa, pd = sc(xw, a, c, b, dst, src)

    cb = 512
    out = pl.pallas_call(
        _combine_body,
        grid=(N_PAD // cb,),
        in_specs=[
            pl.BlockSpec((2, cb, D), lambda i: (0, i, 0)),
            pl.BlockSpec((2, cb), lambda i: (0, i)),
            pl.BlockSpec((1, D), lambda i: (0, 0)),
        ],
        out_specs=pl.BlockSpec((cb, D), lambda i: (i, 0)),
        out_shape=jax.ShapeDtypeStruct((N_PAD, D), f32),
    )(pa, pd, bias.reshape(1, D))

    return out[:N]


# speedup vs baseline: 14.4082x; 1.1421x over previous
import dataclasses
import functools

import jax
import jax.numpy as jnp
from jax import lax
from jax.experimental import pallas as pl
from jax.experimental.pallas import tpu as pltpu
from jax.experimental.pallas import tpu_sc as plsc

N_PAD = 10240
K = 96
CH0 = 68
CH1 = 142
E_PAD = 16 * (CH0 + CH1) * K
E_ALLOC = 327680
STRIPE = N_PAD // 16


def _prep_body(x_ref, w_ref, wac_ref, ea_ref, wbig_ref, xw_ref, ac_ref, b_ref):
    xw = jnp.dot(x_ref[...], w_ref[...], preferred_element_type=jnp.float32)
    xw_ref[...] = xw
    ac_ref[...] = jnp.dot(xw, wac_ref[...], preferred_element_type=jnp.float32)
    b_ref[...] = jnp.dot(ea_ref[...], wbig_ref[...],
                         preferred_element_type=jnp.float32)


def _combine_body(pa_ref, pd_ref, bias_ref, o_ref):
    s = pa_ref[0] + pa_ref[1]
    d = pd_ref[0] + pd_ref[1] + 1e-16
    o_ref[...] = s / d[:, None] + bias_ref[...]


def _sc_body(n_edges,
             xw_hbm, a_hbm, c_hbm, b_hbm, dst_hbm, src_hbm,
             aggr_out, den_out,
             a_buf, c_buf,
             idx_d0, idx_d1, idx_s0, idx_s1, b0, b1, g0, g1, sidx0, sidx1,
             rows0, rows1,
             aggr_sh, den_sh,
             sem_in0, sem_in1, sem_g0, sem_g1, sem_s0, sem_s1):
    cid = lax.axis_index("c")
    sid = lax.axis_index("s")
    wid = sid * 2 + cid
    zeros16 = jnp.zeros((16,), jnp.float32)
    izeros16 = jnp.zeros((16,), jnp.int32)
    idx_d = (idx_d0, idx_d1)
    idx_s = (idx_s0, idx_s1)
    bb = (b0, b1)
    gb = (g0, g1)
    sidx = (sidx0, sidx1)
    rows = (rows0, rows1)
    sem_in = (sem_in0, sem_in1)
    sem_g = (sem_g0, sem_g1)
    sem_s = (sem_s0, sem_s1)

    @pl.loop(0, K // 16)
    def _(i):
        sl = pl.ds(i * 16, 16)
        for q in (0, 1):
            gb[q][sl] = zeros16
            idx_d[q][sl] = izeros16
            idx_s[q][sl] = izeros16
            sidx[q][sl] = izeros16

    @pl.loop(0, K)
    def _(r):
        for cc in range(8):
            sl = pl.ds(cc * 16, 16)
            rows0[r, sl] = zeros16
            rows1[r, sl] = zeros16

    off = sid * STRIPE
    for i in range(STRIPE // K):
        pltpu.sync_copy(gb[0], den_sh.at[pl.ds(off + i * K, K)])
        pltpu.sync_copy(rows0, aggr_sh.at[pl.ds(off + i * K, K)])
    rem = STRIPE - (STRIPE // K) * K
    if rem:
        pltpu.sync_copy(gb[0].at[pl.ds(0, rem)],
                        den_sh.at[pl.ds(off + STRIPE - rem, rem)])
        pltpu.sync_copy(rows0.at[pl.ds(0, rem)],
                        aggr_sh.at[pl.ds(off + STRIPE - rem, rem)])

    pltpu.sync_copy(a_hbm, a_buf)
    pltpu.sync_copy(c_hbm, c_buf)
    plsc.subcore_barrier()

    is0 = cid == 0
    npairs = jnp.where(is0, CH0 // 2, CH1 // 2)
    base0 = jnp.where(is0, sid * (CH0 * K),
                      16 * (CH0 * K) + sid * (CH1 * K))

    def issue_in(q, t):
        base = base0 + t * K
        pltpu.async_copy(dst_hbm.at[pl.ds(base, K)], idx_d[q], sem_in[q])
        pltpu.async_copy(src_hbm.at[pl.ds(base, K)], idx_s[q], sem_in[q])
        pltpu.async_copy(b_hbm.at[pl.ds(base, K)], bb[q], sem_in[q])

    def wait_in(q):
        pltpu.make_async_copy(dst_hbm.at[pl.ds(0, K)], idx_d[q], sem_in[q]).wait()
        pltpu.make_async_copy(src_hbm.at[pl.ds(0, K)], idx_s[q], sem_in[q]).wait()
        pltpu.make_async_copy(b_hbm.at[pl.ds(0, K)], bb[q], sem_in[q]).wait()

    def wait_g(q):
        pltpu.make_async_copy(xw_hbm.at[idx_s[q]], rows[q], sem_g[q]).wait()

    def wait_s(q):
        pltpu.make_async_copy(rows[q], aggr_sh.at[sidx[q]], sem_s[q]).wait()

    pltpu.async_copy(rows1, aggr_sh.at[sidx1], sem_s1, add=True)
    issue_in(0, 0)
    issue_in(1, 1)
    wait_in(0)
    pltpu.async_copy(xw_hbm.at[idx_s0], rows0, sem_g0)

    def phase(q, qn, t):
        base = base0 + t * K

        @pl.loop(0, K // 16)
        def _(j):
            sl = pl.ds(j * 16, 16)
            d16 = idx_d[q][sl]
            s16 = idx_s[q][sl]
            av = plsc.load_gather(a_buf, [d16])
            cv = plsc.load_gather(c_buf, [s16])
            al = av + cv + bb[q][sl]
            al = jnp.where(al > 0, al, al * 0.2)
            al = jnp.minimum(al, 60.0)
            g = jnp.exp(al)
            eid = base + j * 16 + lax.iota(jnp.int32, 16)
            gb[q][sl] = jnp.where(eid < n_edges, g, 0.0)
            sidx[q][sl] = d16

        pltpu.sync_copy(gb[q], den_sh.at[sidx[q]], add=True)
        wait_g(q)
        issue_in(q, t + 2)

        @pl.loop(0, K // 16)
        def _(j):
            g16 = gb[q][pl.ds(j * 16, 16)]
            for l in range(16):
                r = j * 16 + l
                gs = g16[l]
                for cc in range(8):
                    sl2 = pl.ds(cc * 16, 16)
                    rows[q][r, sl2] = rows[q][r, sl2] * gs

        pltpu.async_copy(rows[q], aggr_sh.at[sidx[q]], sem_s[q], add=True)
        wait_in(qn)
        wait_s(qn)
        pltpu.async_copy(xw_hbm.at[idx_s[qn]], rows[qn], sem_g[qn])

    def pair(i, carry):
        phase(0, 1, 2 * i)
        phase(1, 0, 2 * i + 1)
        return carry

    lax.fori_loop(0, npairs, pair, 0)

    wait_in(1)
    wait_g(0)
    wait_s(1)

    plsc.subcore_barrier()
    pltpu.sync_copy(aggr_sh.at[pl.ds(off, STRIPE)],
                    aggr_out.at[cid, pl.ds(off, STRIPE)])
    pltpu.sync_copy(den_sh.at[pl.ds(off, STRIPE)],
                    den_out.at[cid, pl.ds(off, STRIPE)])


def kernel(x, edge_index, edge_attr, W_node, w_att, bias):
    N, D = x.shape
    E, De = edge_attr.shape
    f32 = jnp.float32

    dst = jnp.pad(edge_index[1].astype(jnp.int32), (0, E_ALLOC - E))
    src = jnp.pad(edge_index[0].astype(jnp.int32), (0, E_ALLOC - E))
    wac = jnp.stack([w_att[:D], w_att[D + De:]], axis=1)
    npack = D // De
    blkmask = (jnp.arange(D)[:, None] // De) == jnp.arange(npack)[None, :]
    wbig = jnp.where(blkmask, jnp.tile(w_att[D:D + De], npack)[:, None], 0.0)
    ea_r = edge_attr.reshape(E // npack, D)

    nb = 20
    xb = N_PAD // nb
    eb = E_ALLOC // npack // nb
    xw, ac, b2 = pl.pallas_call(
        _prep_body,
        grid=(nb,),
        in_specs=[
            pl.BlockSpec((xb, D), lambda i: (i, 0)),
            pl.BlockSpec((D, D), lambda i: (0, 0)),
            pl.BlockSpec((D, 2), lambda i: (0, 0)),
            pl.BlockSpec((eb, D), lambda i: (i, 0)),
            pl.BlockSpec((D, npack), lambda i: (0, 0)),
        ],
        out_specs=[
            pl.BlockSpec((xb, D), lambda i: (i, 0)),
            pl.BlockSpec((xb, 2), lambda i: (i, 0)),
            pl.BlockSpec((eb, npack), lambda i: (i, 0)),
        ],
        out_shape=[
            jax.ShapeDtypeStruct((N_PAD, D), f32),
            jax.ShapeDtypeStruct((N_PAD, 2), f32),
            jax.ShapeDtypeStruct((E_ALLOC // npack, npack), f32),
        ],
    )(x, W_node, wac, ea_r, wbig)

    a = ac[:, 0]
    c = ac[:, 1]
    b = b2.reshape(E_ALLOC)

    mesh = plsc.VectorSubcoreMesh(core_axis_name="c", subcore_axis_name="s")
    cp = pltpu.CompilerParams()
    if "needs_layout_passes" in pltpu.CompilerParams.__dataclass_fields__:
        cp = dataclasses.replace(cp, needs_layout_passes=False)
    sc = pl.kernel(
        functools.partial(_sc_body, E),
        out_type=[
            jax.ShapeDtypeStruct((2, N_PAD, D), f32),
            jax.ShapeDtypeStruct((2, N_PAD), f32),
        ],
        mesh=mesh,
        scratch_types=[
            pltpu.VMEM((N_PAD,), f32),
            pltpu.VMEM((N_PAD,), f32),
            pltpu.VMEM((K,), jnp.int32),
            pltpu.VMEM((K,), jnp.int32),
            pltpu.VMEM((K,), jnp.int32),
            pltpu.VMEM((K,), jnp.int32),
            pltpu.VMEM((K,), f32),
            pltpu.VMEM((K,), f32),
            pltpu.VMEM((K,), f32),
            pltpu.VMEM((K,), f32),
            pltpu.VMEM((K,), jnp.int32),
            pltpu.VMEM((K,), jnp.int32),
            pltpu.VMEM((K, D), f32),
            pltpu.VMEM((K, D), f32),
            pltpu.VMEM_SHARED((N_PAD, D), f32),
            pltpu.VMEM_SHARED((N_PAD,), f32),
            pltpu.SemaphoreType.DMA,
            pltpu.SemaphoreType.DMA,
            pltpu.SemaphoreType.DMA,
            pltpu.SemaphoreType.DMA,
            pltpu.SemaphoreType.DMA,
            pltpu.SemaphoreType.DMA,
        ],
        compiler_params=cp,
    )
    pa, pd = sc(xw, a, c, b, dst, src)

    cb = 512
    out = pl.pallas_call(
        _combine_body,
        grid=(N_PAD // cb,),
        in_specs=[
            pl.BlockSpec((2, cb, D), lambda i: (0, i, 0)),
            pl.BlockSpec((2, cb), lambda i: (0, i)),
            pl.BlockSpec((1, D), lambda i: (0, 0)),
        ],
        out_specs=pl.BlockSpec((cb, D), lambda i: (i, 0)),
        out_shape=jax.ShapeDtypeStruct((N_PAD, D), f32),
    )(pa, pd, bias.reshape(1, D))

    return out[:N]
